# baseline (speedup 1.0000x reference)
import functools

import jax
import jax.numpy as jnp
from jax import lax
from jax.experimental import pallas as pl
from jax.experimental.pallas import tpu as pltpu
from jax.experimental.pallas import tpu_sc as plsc

BS = 8
NP = 1024
K = 20
NSHARD = 1
BH = BS // NSHARD
NH = BH * NP
NWORKERS = 32
IW = 64


def _knn_head(x_ref, xt_ref, et_ref, w1e1_ref, w2e1_ref, be1_ref, wc1_ref,
              bc1_ref, wc2_ref, bc2_ref, idx_ref, y1_ref, y2_ref, e_ref,
              e2_ref):
    b = pl.program_id(0)
    x = x_ref[0]
    xt = xt_ref[0]
    inner = jnp.dot(xt, x, preferred_element_type=jnp.float32)
    sqc = jnp.sum(xt * xt, axis=1, keepdims=True)
    sqr = jnp.sum(x * x, axis=0, keepdims=True)
    dist = sqc + sqr - 2.0 * inner
    iota = lax.broadcasted_iota(jnp.int32, (NP, NP), 1)
    cols = []
    for _ in range(K):
        mn = jnp.min(dist, axis=1, keepdims=True)
        am = jnp.min(jnp.where(dist == mn, iota, NP + 1), axis=1,
                     keepdims=True)
        cols.append(am)
        dist = jnp.where(iota == am, jnp.float32(jnp.inf), dist)
    idx_ref[0] = jnp.concatenate(cols, axis=1) + b * NP

    y1_ref[0] = (jnp.dot(xt, w1e1_ref[...], preferred_element_type=jnp.float32)
                 + be1_ref[...])
    y2_ref[0] = jnp.dot(xt, w2e1_ref[...], preferred_element_type=jnp.float32)
    e = jax.nn.relu(jnp.dot(et_ref[0], wc1_ref[...],
                            preferred_element_type=jnp.float32) + bc1_ref[...])
    e_ref[0] = e
    e2_ref[0] = jax.nn.relu(jnp.dot(e, wc2_ref[...],
                                    preferred_element_type=jnp.float32)
                            + bc2_ref[...])


def _mid_layer(y1_ref, m_ref, w1_ref, w2_ref, b_ref, h_ref, y1o_ref, y2o_ref):
    h = jax.nn.relu(y1_ref[...] + m_ref[...])
    h_ref[...] = h
    y1o_ref[...] = (jnp.dot(h, w1_ref[...], preferred_element_type=jnp.float32)
                    + b_ref[...])
    y2o_ref[...] = jnp.dot(h, w2_ref[...], preferred_element_type=jnp.float32)


def _tail(y1_ref, m_ref, e2_ref, wt1_ref, bt1_ref, wt2_ref, bt2_ref, wr1_ref,
          br1_ref, wr2_ref, br2_ref, tg_ref, rg_ref):
    h3 = jax.nn.relu(y1_ref[0] + m_ref[0])
    fusion = jnp.concatenate([h3, e2_ref[0]], axis=1)
    t = jax.nn.relu(jnp.dot(fusion, wt1_ref[...],
                            preferred_element_type=jnp.float32) + bt1_ref[...])
    t = jax.nn.relu(jnp.dot(t, wt2_ref[...],
                            preferred_element_type=jnp.float32) + bt2_ref[...])
    tg_ref[0] = jnp.sum(t, axis=0, keepdims=True) * (1.0 / NP)
    r = jax.nn.relu(jnp.dot(fusion, wr1_ref[...],
                            preferred_element_type=jnp.float32) + br1_ref[...])
    r = jax.nn.relu(jnp.dot(r, wr2_ref[...],
                            preferred_element_type=jnp.float32) + br2_ref[...])
    rg_ref[0] = jnp.sum(r, axis=0, keepdims=True) * (1.0 / NP)


def _rep(shape):
    nd = len(shape)
    return pl.BlockSpec(shape, lambda b: (0,) * nd)


def _gather_max(table, idx2d, c):
    n = table.shape[0]
    gpw = n // NWORKERS
    GPC = 32 if c <= 64 else 16
    RPC = GPC * K
    IROWS = RPC // IW
    nchunk = gpw // GPC
    mesh = plsc.VectorSubcoreMesh(core_axis_name="c", subcore_axis_name="s")

    @functools.partial(
        pl.kernel,
        out_type=jax.ShapeDtypeStruct((n, c), jnp.float32),
        mesh=mesh,
        scratch_types=[
            pltpu.VMEM((nchunk * IROWS, IW), jnp.int32),
            pltpu.VMEM((2, RPC, c), jnp.float32),
            pltpu.VMEM((2, GPC, c), jnp.float32),
            pltpu.SemaphoreType.DMA,
            pltpu.SemaphoreType.DMA,
            pltpu.SemaphoreType.DMA,
        ],
        compiler_params=pltpu.CompilerParams(use_tc_tiling_on_sc=False),
    )
    def gmax(table_hbm, idx_hbm, out_hbm, idx_v, rows_v, out_v, g0, g1, so):
        wid = lax.axis_index("s") * 2 + lax.axis_index("c")
        nrow = nchunk * IROWS
        gsem = (g0, g1)
        pltpu.sync_copy(idx_hbm.at[pl.ds(wid * nrow, nrow)], idx_v)

        def fire(chunk, buf):
            for i in range(IROWS):
                pltpu.async_copy(
                    table_hbm.at[idx_v.at[chunk * IROWS + i]],
                    rows_v.at[buf].at[pl.ds(i * IW, IW)], gsem[buf])

        fire(0, 0)
        fire(1, 1)

        def do_chunk(chunk, buf, first):
            pltpu.make_async_copy(
                table_hbm.at[pl.ds(0, RPC)], rows_v.at[buf], gsem[buf]).wait()

            @pl.when(jnp.logical_not(first))
            def _():
                pltpu.make_async_copy(
                    table_hbm.at[pl.ds(0, GPC)], out_v.at[buf], so).wait()

            def body(g2, carry):
                for u in range(2):
                    g = g2 * 2 + u
                    for lc in range(c // 16):
                        acc = rows_v[buf, g * K, pl.ds(lc * 16, 16)]
                        for j in range(1, K):
                            acc = jnp.maximum(
                                acc, rows_v[buf, g * K + j, pl.ds(lc * 16, 16)])
                        out_v[buf, g, pl.ds(lc * 16, 16)] = acc
                return carry

            lax.fori_loop(0, GPC // 2, body, 0)

            @pl.when(chunk + 2 < nchunk)
            def _():
                fire(chunk + 2, buf)

            pltpu.async_copy(
                out_v.at[buf],
                out_hbm.at[pl.ds(wid * gpw + chunk * GPC, GPC)], so)

        def pair(p, carry):
            do_chunk(2 * p, 0, p == 0)
            do_chunk(2 * p + 1, 1, p == 0)
            return carry

        lax.fori_loop(0, nchunk // 2, pair, 0)
        pltpu.make_async_copy(
            table_hbm.at[pl.ds(0, GPC)], out_v.at[0], so).wait()
        pltpu.make_async_copy(
            table_hbm.at[pl.ds(0, GPC)], out_v.at[1], so).wait()

    return gmax(table, idx2d)


def _head_call(x_h, xt_h, et_h, w1e1, w2e1, be1, wc1t, bc1, wc2t, bc2):
    f32 = jnp.float32
    return pl.pallas_call(
        _knn_head,
        grid=(BH,),
        in_specs=[
            pl.BlockSpec((1, 3, NP), lambda b: (b, 0, 0)),
            pl.BlockSpec((1, NP, 3), lambda b: (b, 0, 0)),
            pl.BlockSpec((1, NP, 32), lambda b: (b, 0, 0)),
            _rep((3, 64)), _rep((3, 64)), _rep((1, 64)),
            _rep((32, 64)), _rep((1, 64)),
            _rep((64, 128)), _rep((1, 128)),
        ],
        out_specs=[
            pl.BlockSpec((1, NP, K), lambda b: (b, 0, 0)),
            pl.BlockSpec((1, NP, 64), lambda b: (b, 0, 0)),
            pl.BlockSpec((1, NP, 64), lambda b: (b, 0, 0)),
            pl.BlockSpec((1, NP, 64), lambda b: (b, 0, 0)),
            pl.BlockSpec((1, NP, 128), lambda b: (b, 0, 0)),
        ],
        out_shape=[
            jax.ShapeDtypeStruct((BH, NP, K), jnp.int32),
            jax.ShapeDtypeStruct((BH, NP, 64), f32),
            jax.ShapeDtypeStruct((BH, NP, 64), f32),
            jax.ShapeDtypeStruct((BH, NP, 64), f32),
            jax.ShapeDtypeStruct((BH, NP, 128), f32),
        ],
    )(x_h, xt_h, et_h, w1e1, w2e1, be1, wc1t, bc1, wc2t, bc2)


def _mid_call(y1, m, w1, w2, b, co):
    f32 = jnp.float32
    ci = w1.shape[0]
    return pl.pallas_call(
        _mid_layer,
        in_specs=[pl.BlockSpec((NH, ci), lambda: (0, 0)),
                  pl.BlockSpec((NH, ci), lambda: (0, 0)),
                  pl.BlockSpec((ci, co), lambda: (0, 0)),
                  pl.BlockSpec((ci, co), lambda: (0, 0)),
                  pl.BlockSpec((1, co), lambda: (0, 0))],
        out_specs=[pl.BlockSpec((NH, ci), lambda: (0, 0)),
                   pl.BlockSpec((NH, co), lambda: (0, 0)),
                   pl.BlockSpec((NH, co), lambda: (0, 0))],
        out_shape=[jax.ShapeDtypeStruct((NH, ci), f32),
                   jax.ShapeDtypeStruct((NH, co), f32),
                   jax.ShapeDtypeStruct((NH, co), f32)],
    )(y1, m, w1, w2, b.reshape(1, co))


def _tail_call(y1_3, m3, e2, wt1t, bt1, wt2t, bt2, wr1t, br1, wr2t, br2):
    f32 = jnp.float32
    return pl.pallas_call(
        _tail,
        grid=(BH,),
        in_specs=[
            pl.BlockSpec((1, NP, 128), lambda b: (b, 0, 0)),
            pl.BlockSpec((1, NP, 128), lambda b: (b, 0, 0)),
            pl.BlockSpec((1, NP, 128), lambda b: (b, 0, 0)),
            _rep((256, 256)), _rep((1, 256)),
            _rep((256, 1024)), _rep((1, 1024)),
            _rep((256, 256)), _rep((1, 256)),
            _rep((256, 1024)), _rep((1, 1024)),
        ],
        out_specs=[
            pl.BlockSpec((1, 1, 1024), lambda b: (b, 0, 0)),
            pl.BlockSpec((1, 1, 1024), lambda b: (b, 0, 0)),
        ],
        out_shape=[
            jax.ShapeDtypeStruct((BH, 1, 1024), f32),
            jax.ShapeDtypeStruct((BH, 1, 1024), f32),
        ],
    )(y1_3.reshape(BH, NP, 128), m3.reshape(BH, NP, 128),
      e2.reshape(BH, NP, 128), wt1t, bt1.reshape(1, 256), wt2t,
      bt2.reshape(1, 1024), wr1t, br1.reshape(1, 256), wr2t,
      br2.reshape(1, 1024))


def kernel(x, emb, W_e1, b_e1, W_e2, b_e2, W_e3, b_e3, W_c1, b_c1, W_c2, b_c2,
           W_t1, b_t1, W_t2, b_t2, W_r1, b_r1, W_r2, b_r2):
    xt = jnp.transpose(x, (0, 2, 1))
    et = jnp.transpose(emb, (0, 2, 1))

    def split(w):
        c = w.shape[1] // 2
        return (w[:, :c] - w[:, c:]).T, w[:, c:].T

    w1e1, w2e1 = split(W_e1)
    w1e2, w2e2 = split(W_e2)
    w1e3, w2e3 = split(W_e3)
    be1 = b_e1.reshape(1, 64)
    wc1t, bc1 = W_c1.T, b_c1.reshape(1, 64)
    wc2t, bc2 = W_c2.T, b_c2.reshape(1, 128)
    wt1t, wt2t, wr1t, wr2t = W_t1.T, W_t2.T, W_r1.T, W_r2.T

    heads = []
    for s in range(NSHARD):
        sl = slice(s * BH, (s + 1) * BH)
        heads.append(_head_call(x[sl], xt[sl], et[sl], w1e1, w2e1, be1,
                                wc1t, bc1, wc2t, bc2))

    outs = []
    for s in range(NSHARD):
        nn_idx, y1_1, y2_1, e, e2 = heads[s]
        idx2d = nn_idx.reshape(NH * K // IW, IW)
        y1_1 = y1_1.reshape(NH, 64)
        y2_1 = y2_1.reshape(NH, 64)
        e = e.reshape(NH, 64)
        e2 = e2.reshape(NH, 128)

        m1 = _gather_max(y2_1, idx2d, 64)
        _, y1_2, y2_2 = _mid_call(y1_1, m1, w1e2, w2e2, b_e2, 64)
        m2 = _gather_max(y2_2, idx2d, 64)
        h2, y1_3, y2_3 = _mid_call(y1_2, m2, w1e3, w2e3, b_e3, 128)
        m3 = _gather_max(y2_3, idx2d, 128)
        tg, rg = _tail_call(y1_3, m3, e2, wt1t, b_t1, wt2t, b_t2,
                            wr1t, b_r1, wr2t, b_r2)
        outs.append((h2, e, tg[:, 0], rg[:, 0]))

    h2 = jnp.concatenate([o[0].reshape(BH, NP, 64) for o in outs], axis=0)
    e = jnp.concatenate([o[1].reshape(BH, NP, 64) for o in outs], axis=0)
    tg = jnp.concatenate([o[2] for o in outs], axis=0)
    rg = jnp.concatenate([o[3] for o in outs], axis=0)

    pf = jnp.transpose(jnp.concatenate([h2, e], axis=2), (0, 2, 1))
    t_feat = jnp.concatenate(
        [pf, jnp.broadcast_to(tg[:, :, None], (BS, 1024, NP))], axis=1)
    return (t_feat, rg[:, :, None])


# speedup vs baseline: 1.1430x; 1.1430x over previous
import functools

import jax
import jax.numpy as jnp
from jax import lax
from jax.experimental import pallas as pl
from jax.experimental.pallas import tpu as pltpu
from jax.experimental.pallas import tpu_sc as plsc

BS = 8
NP = 1024
K = 20
NSHARD = 1
BH = BS // NSHARD
NH = BH * NP
NWORKERS = 32
IW = 64


def _knn_head(x_ref, xt_ref, et_ref, w1e1_ref, w2e1_ref, be1_ref, wc1_ref,
              bc1_ref, wc2_ref, bc2_ref, idx_ref, y1_ref, y2_ref, e_ref,
              e2_ref):
    b = pl.program_id(0)
    x = x_ref[0]
    xt = xt_ref[0]
    inner = jnp.dot(xt, x, preferred_element_type=jnp.float32)
    sqc = jnp.sum(xt * xt, axis=1, keepdims=True)
    sqr = jnp.sum(x * x, axis=0, keepdims=True)
    dist = sqc + sqr - 2.0 * inner
    iota = lax.broadcasted_iota(jnp.int32, (NP, NP), 1)
    cols = []
    for p in range(K):
        am = jnp.argmin(dist, axis=1).astype(jnp.int32)[:, None]
        cols.append(am)
        if p + 1 < K:
            dist = jnp.where(iota == am, jnp.float32(jnp.inf), dist)
    idx_ref[0] = jnp.concatenate(cols, axis=1) + b * NP

    y1_ref[0] = (jnp.dot(xt, w1e1_ref[...], preferred_element_type=jnp.float32)
                 + be1_ref[...])
    y2_ref[0] = jnp.dot(xt, w2e1_ref[...], preferred_element_type=jnp.float32)
    e = jax.nn.relu(jnp.dot(et_ref[0], wc1_ref[...],
                            preferred_element_type=jnp.float32) + bc1_ref[...])
    e_ref[0] = e
    e2_ref[0] = jax.nn.relu(jnp.dot(e, wc2_ref[...],
                                    preferred_element_type=jnp.float32)
                            + bc2_ref[...])


def _mid_layer(y1_ref, m_ref, w1_ref, w2_ref, b_ref, h_ref, y1o_ref, y2o_ref):
    h = jax.nn.relu(y1_ref[...] + m_ref[...])
    h_ref[...] = h
    y1o_ref[...] = (jnp.dot(h, w1_ref[...], preferred_element_type=jnp.float32)
                    + b_ref[...])
    y2o_ref[...] = jnp.dot(h, w2_ref[...], preferred_element_type=jnp.float32)


def _tail(y1_ref, m_ref, e2_ref, wt1_ref, bt1_ref, wt2_ref, bt2_ref, wr1_ref,
          br1_ref, wr2_ref, br2_ref, tg_ref, rg_ref):
    h3 = jax.nn.relu(y1_ref[0] + m_ref[0])
    fusion = jnp.concatenate([h3, e2_ref[0]], axis=1)
    t = jax.nn.relu(jnp.dot(fusion, wt1_ref[...],
                            preferred_element_type=jnp.float32) + bt1_ref[...])
    t = jax.nn.relu(jnp.dot(t, wt2_ref[...],
                            preferred_element_type=jnp.float32) + bt2_ref[...])
    tg_ref[0] = jnp.sum(t, axis=0, keepdims=True) * (1.0 / NP)
    r = jax.nn.relu(jnp.dot(fusion, wr1_ref[...],
                            preferred_element_type=jnp.float32) + br1_ref[...])
    r = jax.nn.relu(jnp.dot(r, wr2_ref[...],
                            preferred_element_type=jnp.float32) + br2_ref[...])
    rg_ref[0] = jnp.sum(r, axis=0, keepdims=True) * (1.0 / NP)


def _rep(shape):
    nd = len(shape)
    return pl.BlockSpec(shape, lambda b: (0,) * nd)


def _gather_max(table, idx2d, c):
    n = table.shape[0]
    gpw = n // NWORKERS
    GPC = 32 if c <= 64 else 16
    RPC = GPC * K
    IROWS = RPC // IW
    nchunk = gpw // GPC
    mesh = plsc.VectorSubcoreMesh(core_axis_name="c", subcore_axis_name="s")

    @functools.partial(
        pl.kernel,
        out_type=jax.ShapeDtypeStruct((n, c), jnp.float32),
        mesh=mesh,
        scratch_types=[
            pltpu.VMEM((nchunk * IROWS, IW), jnp.int32),
            pltpu.VMEM((2, RPC, c), jnp.float32),
            pltpu.VMEM((2, GPC, c), jnp.float32),
            pltpu.SemaphoreType.DMA,
            pltpu.SemaphoreType.DMA,
            pltpu.SemaphoreType.DMA,
        ],
        compiler_params=pltpu.CompilerParams(use_tc_tiling_on_sc=False),
    )
    def gmax(table_hbm, idx_hbm, out_hbm, idx_v, rows_v, out_v, g0, g1, so):
        wid = lax.axis_index("s") * 2 + lax.axis_index("c")
        nrow = nchunk * IROWS
        gsem = (g0, g1)
        pltpu.sync_copy(idx_hbm.at[pl.ds(wid * nrow, nrow)], idx_v)

        def fire(chunk, buf):
            for i in range(IROWS):
                pltpu.async_copy(
                    table_hbm.at[idx_v.at[chunk * IROWS + i]],
                    rows_v.at[buf].at[pl.ds(i * IW, IW)], gsem[buf])

        fire(0, 0)
        fire(1, 1)

        def do_chunk(chunk, buf, first):
            pltpu.make_async_copy(
                table_hbm.at[pl.ds(0, RPC)], rows_v.at[buf], gsem[buf]).wait()

            @pl.when(jnp.logical_not(first))
            def _():
                pltpu.make_async_copy(
                    table_hbm.at[pl.ds(0, GPC)], out_v.at[buf], so).wait()

            def body(g2, carry):
                for u in range(2):
                    g = g2 * 2 + u
                    for lc in range(c // 16):
                        acc = rows_v[buf, g * K, pl.ds(lc * 16, 16)]
                        for j in range(1, K):
                            acc = jnp.maximum(
                                acc, rows_v[buf, g * K + j, pl.ds(lc * 16, 16)])
                        out_v[buf, g, pl.ds(lc * 16, 16)] = acc
                return carry

            lax.fori_loop(0, GPC // 2, body, 0)

            @pl.when(chunk + 2 < nchunk)
            def _():
                fire(chunk + 2, buf)

            pltpu.async_copy(
                out_v.at[buf],
                out_hbm.at[pl.ds(wid * gpw + chunk * GPC, GPC)], so)

        def pair(p, carry):
            do_chunk(2 * p, 0, p == 0)
            do_chunk(2 * p + 1, 1, p == 0)
            return carry

        lax.fori_loop(0, nchunk // 2, pair, 0)
        pltpu.make_async_copy(
            table_hbm.at[pl.ds(0, GPC)], out_v.at[0], so).wait()
        pltpu.make_async_copy(
            table_hbm.at[pl.ds(0, GPC)], out_v.at[1], so).wait()

    return gmax(table, idx2d)


def _head_call(x_h, xt_h, et_h, w1e1, w2e1, be1, wc1t, bc1, wc2t, bc2):
    f32 = jnp.float32
    return pl.pallas_call(
        _knn_head,
        grid=(BH,),
        in_specs=[
            pl.BlockSpec((1, 3, NP), lambda b: (b, 0, 0)),
            pl.BlockSpec((1, NP, 3), lambda b: (b, 0, 0)),
            pl.BlockSpec((1, NP, 32), lambda b: (b, 0, 0)),
            _rep((3, 64)), _rep((3, 64)), _rep((1, 64)),
            _rep((32, 64)), _rep((1, 64)),
            _rep((64, 128)), _rep((1, 128)),
        ],
        out_specs=[
            pl.BlockSpec((1, NP, K), lambda b: (b, 0, 0)),
            pl.BlockSpec((1, NP, 64), lambda b: (b, 0, 0)),
            pl.BlockSpec((1, NP, 64), lambda b: (b, 0, 0)),
            pl.BlockSpec((1, NP, 64), lambda b: (b, 0, 0)),
            pl.BlockSpec((1, NP, 128), lambda b: (b, 0, 0)),
        ],
        out_shape=[
            jax.ShapeDtypeStruct((BH, NP, K), jnp.int32),
            jax.ShapeDtypeStruct((BH, NP, 64), f32),
            jax.ShapeDtypeStruct((BH, NP, 64), f32),
            jax.ShapeDtypeStruct((BH, NP, 64), f32),
            jax.ShapeDtypeStruct((BH, NP, 128), f32),
        ],
    )(x_h, xt_h, et_h, w1e1, w2e1, be1, wc1t, bc1, wc2t, bc2)


def _mid_call(y1, m, w1, w2, b, co):
    f32 = jnp.float32
    ci = w1.shape[0]
    return pl.pallas_call(
        _mid_layer,
        in_specs=[pl.BlockSpec((NH, ci), lambda: (0, 0)),
                  pl.BlockSpec((NH, ci), lambda: (0, 0)),
                  pl.BlockSpec((ci, co), lambda: (0, 0)),
                  pl.BlockSpec((ci, co), lambda: (0, 0)),
                  pl.BlockSpec((1, co), lambda: (0, 0))],
        out_specs=[pl.BlockSpec((NH, ci), lambda: (0, 0)),
                   pl.BlockSpec((NH, co), lambda: (0, 0)),
                   pl.BlockSpec((NH, co), lambda: (0, 0))],
        out_shape=[jax.ShapeDtypeStruct((NH, ci), f32),
                   jax.ShapeDtypeStruct((NH, co), f32),
                   jax.ShapeDtypeStruct((NH, co), f32)],
    )(y1, m, w1, w2, b.reshape(1, co))


def _tail_call(y1_3, m3, e2, wt1t, bt1, wt2t, bt2, wr1t, br1, wr2t, br2):
    f32 = jnp.float32
    return pl.pallas_call(
        _tail,
        grid=(BH,),
        in_specs=[
            pl.BlockSpec((1, NP, 128), lambda b: (b, 0, 0)),
            pl.BlockSpec((1, NP, 128), lambda b: (b, 0, 0)),
            pl.BlockSpec((1, NP, 128), lambda b: (b, 0, 0)),
            _rep((256, 256)), _rep((1, 256)),
            _rep((256, 1024)), _rep((1, 1024)),
            _rep((256, 256)), _rep((1, 256)),
            _rep((256, 1024)), _rep((1, 1024)),
        ],
        out_specs=[
            pl.BlockSpec((1, 1, 1024), lambda b: (b, 0, 0)),
            pl.BlockSpec((1, 1, 1024), lambda b: (b, 0, 0)),
        ],
        out_shape=[
            jax.ShapeDtypeStruct((BH, 1, 1024), f32),
            jax.ShapeDtypeStruct((BH, 1, 1024), f32),
        ],
    )(y1_3.reshape(BH, NP, 128), m3.reshape(BH, NP, 128),
      e2.reshape(BH, NP, 128), wt1t, bt1.reshape(1, 256), wt2t,
      bt2.reshape(1, 1024), wr1t, br1.reshape(1, 256), wr2t,
      br2.reshape(1, 1024))


def kernel(x, emb, W_e1, b_e1, W_e2, b_e2, W_e3, b_e3, W_c1, b_c1, W_c2, b_c2,
           W_t1, b_t1, W_t2, b_t2, W_r1, b_r1, W_r2, b_r2):
    xt = jnp.transpose(x, (0, 2, 1))
    et = jnp.transpose(emb, (0, 2, 1))

    def split(w):
        c = w.shape[1] // 2
        return (w[:, :c] - w[:, c:]).T, w[:, c:].T

    w1e1, w2e1 = split(W_e1)
    w1e2, w2e2 = split(W_e2)
    w1e3, w2e3 = split(W_e3)
    be1 = b_e1.reshape(1, 64)
    wc1t, bc1 = W_c1.T, b_c1.reshape(1, 64)
    wc2t, bc2 = W_c2.T, b_c2.reshape(1, 128)
    wt1t, wt2t, wr1t, wr2t = W_t1.T, W_t2.T, W_r1.T, W_r2.T

    heads = []
    for s in range(NSHARD):
        sl = slice(s * BH, (s + 1) * BH)
        heads.append(_head_call(x[sl], xt[sl], et[sl], w1e1, w2e1, be1,
                                wc1t, bc1, wc2t, bc2))

    outs = []
    for s in range(NSHARD):
        nn_idx, y1_1, y2_1, e, e2 = heads[s]
        idx2d = nn_idx.reshape(NH * K // IW, IW)
        y1_1 = y1_1.reshape(NH, 64)
        y2_1 = y2_1.reshape(NH, 64)
        e = e.reshape(NH, 64)
        e2 = e2.reshape(NH, 128)

        m1 = _gather_max(y2_1, idx2d, 64)
        _, y1_2, y2_2 = _mid_call(y1_1, m1, w1e2, w2e2, b_e2, 64)
        m2 = _gather_max(y2_2, idx2d, 64)
        h2, y1_3, y2_3 = _mid_call(y1_2, m2, w1e3, w2e3, b_e3, 128)
        m3 = _gather_max(y2_3, idx2d, 128)
        tg, rg = _tail_call(y1_3, m3, e2, wt1t, b_t1, wt2t, b_t2,
                            wr1t, b_r1, wr2t, b_r2)
        outs.append((h2, e, tg[:, 0], rg[:, 0]))

    h2 = jnp.concatenate([o[0].reshape(BH, NP, 64) for o in outs], axis=0)
    e = jnp.concatenate([o[1].reshape(BH, NP, 64) for o in outs], axis=0)
    tg = jnp.concatenate([o[2] for o in outs], axis=0)
    rg = jnp.concatenate([o[3] for o in outs], axis=0)

    pf = jnp.transpose(jnp.concatenate([h2, e], axis=2), (0, 2, 1))
    t_feat = jnp.concatenate(
        [pf, jnp.broadcast_to(tg[:, :, None], (BS, 1024, NP))], axis=1)
    return (t_feat, rg[:, :, None])
